# Initial kernel scaffold; baseline (speedup 1.0000x reference)
import jax
import jax.numpy as jnp
from jax.experimental import pallas as pl


def kernel(x, mean, diag, covm):
    raise NotImplementedError("write your pallas kernel here")


# speedup vs baseline: 127.9938x; 127.9938x over previous
import functools

import jax
import jax.numpy as jnp
import numpy as np
from jax import lax
from jax.experimental import pallas as pl
from jax.experimental.pallas import tpu as pltpu
from jax.experimental.pallas import tpu_sc as plsc

DIM = 16
RANK = 2
LW = 128


def _sc_gather(idx3d, mean, diag, covm2):
    nw, rows_per_w, _ = idx3d.shape
    info = plsc.get_sparse_core_info()
    assert nw == info.num_cores * info.num_subcores
    npts = nw * rows_per_w * LW
    mesh = plsc.VectorSubcoreMesh(core_axis_name="c", subcore_axis_name="s")

    @functools.partial(
        pl.kernel,
        out_type=(
            jax.ShapeDtypeStruct((npts, DIM), jnp.float32),
            jax.ShapeDtypeStruct((npts, DIM), jnp.float32),
            jax.ShapeDtypeStruct((npts, 2 * DIM), jnp.float32),
        ),
        mesh=mesh,
        scratch_types=[
            pltpu.VMEM((rows_per_w, LW), jnp.int32),
            pltpu.VMEM((LW, DIM), jnp.float32),
            pltpu.VMEM((LW, DIM), jnp.float32),
            pltpu.VMEM((LW, 2 * DIM), jnp.float32),
            pltpu.SemaphoreType.DMA,
        ],
        compiler_params=pltpu.CompilerParams(use_tc_tiling_on_sc=False),
    )
    def gather_k(idx_hbm, mean_hbm, diag_hbm, covm_hbm, gm_hbm, gd_hbm, gc_hbm,
                 idx_v, mb, db, cb, sem):
        wid = lax.axis_index("s") * info.num_cores + lax.axis_index("c")
        r0 = wid * rows_per_w
        pltpu.sync_copy(idx_hbm.at[wid], idx_v)
        for j in range(rows_per_w):
            row = idx_v.at[j]
            c1 = pltpu.async_copy(mean_hbm.at[row], mb, sem)
            c2 = pltpu.async_copy(diag_hbm.at[row], db, sem)
            c3 = pltpu.async_copy(covm_hbm.at[row], cb, sem)
            c1.wait()
            c2.wait()
            c3.wait()
            base = (r0 + j) * LW
            pltpu.sync_copy(mb, gm_hbm.at[pl.ds(base, LW)])
            pltpu.sync_copy(db, gd_hbm.at[pl.ds(base, LW)])
            pltpu.sync_copy(cb, gc_hbm.at[pl.ds(base, LW)])

    return gather_k(idx3d, mean, diag, covm2)


def _kl_tc(gm, gd, gu, gv, batch, k):
    bblk = 64
    grid = batch // bblk

    def body(m_ref, d_ref, u_ref, v_ref, o_ref):
        mu = m_ref[...].reshape(bblk, k, DIM)
        dd = jnp.clip(d_ref[...].reshape(bblk, k, DIM), 0.01, np.inf)
        u = u_ref[...].reshape(bblk, k, DIM)
        v = v_ref[...].reshape(bblk, k, DIM)
        e = 1.0 / dd
        tu = u * e
        tv = v * e
        m00 = 1.0 + jnp.sum(u * tu, -1)
        m01 = jnp.sum(u * tv, -1)
        m11 = 1.0 + jnp.sum(v * tv, -1)
        det = m00 * m11 - m01 * m01
        ld_full = jnp.log(det) + jnp.sum(jnp.log(dd), -1)

        e1 = e[:, 1:]
        tu1 = tu[:, 1:]
        tv1 = tv[:, 1:]
        n00, n01, n11, ndet = m00[:, 1:], m01[:, 1:], m11[:, 1:], det[:, 1:]

        def qf(a, b):
            return (n11 * a * a - 2.0 * n01 * a * b + n00 * b * b) / ndet

        d0 = dd[:, 0:1]
        c0u = u[:, 0:1]
        c0v = v[:, 0:1]
        term_diag = jnp.sum(d0 * e1, -1)
        g_uu = jnp.sum(tu1 * tu1 * d0, -1)
        g_uv = jnp.sum(tu1 * tv1 * d0, -1)
        g_vv = jnp.sum(tv1 * tv1 * d0, -1)
        gterm = (n11 * g_uu - 2.0 * n01 * g_uv + n00 * g_vv) / ndet
        s = jnp.sum((c0u * c0u + c0v * c0v) * e1, -1)
        a_uu = jnp.sum(c0u * tu1, -1)
        a_uv = jnp.sum(c0u * tv1, -1)
        a_vu = jnp.sum(c0v * tu1, -1)
        a_vv = jnp.sum(c0v * tv1, -1)
        low = qf(a_uu, a_uv) + qf(a_vu, a_vv)
        tr = term_diag - gterm + s - low

        delta = mu[:, 1:] - mu[:, 0:1]
        dq = jnp.sum(delta * delta * e1, -1)
        p_u = jnp.sum(delta * tu1, -1)
        p_v = jnp.sum(delta * tv1, -1)
        quad = dq - qf(p_u, p_v)

        kl = 0.5 * (tr + quad - DIM + ld_full[:, 1:] - ld_full[:, 0:1])
        o_ref[...] = kl

    spec = pl.BlockSpec((bblk * k, DIM), lambda i: (i, 0))
    return pl.pallas_call(
        body,
        grid=(grid,),
        in_specs=[spec, spec, spec, spec],
        out_specs=pl.BlockSpec((bblk, k - 1), lambda i: (i, 0)),
        out_shape=jax.ShapeDtypeStruct((batch, k - 1), jnp.float32),
    )(gm, gd, gu, gv)


def kernel(x, mean, diag, covm):
    batch, k = x.shape
    nw = 32
    idx3d = x.reshape(nw, -1, LW)
    covm2 = covm.reshape(covm.shape[0], DIM * RANK)
    gm, gd, gc = _sc_gather(idx3d, mean, diag, covm2)
    gu = gc[:, 0::2]
    gv = gc[:, 1::2]
    return _kl_tc(gm, gd, gu, gv, batch, k)


# speedup vs baseline: 200.6125x; 1.5674x over previous
import functools

import jax
import jax.numpy as jnp
import numpy as np
from jax import lax
from jax.experimental import pallas as pl
from jax.experimental.pallas import tpu as pltpu
from jax.experimental.pallas import tpu_sc as plsc

DIM = 16
RANK = 2
LW = 128
CHUNK_B = 32
LN2 = 0.6931471805599453


def _vlog(x):
    bits = plsc.bitcast(x, jnp.int32)
    e = jnp.right_shift(bits, 23) - 127
    m = plsc.bitcast(
        jnp.bitwise_or(jnp.bitwise_and(bits, 0x007FFFFF), 0x3F800000),
        jnp.float32)
    big = m > 1.4142135623730951
    m = jnp.where(big, m * 0.5, m)
    e = jnp.where(big, e + 1, e)
    s = (m - 1.0) / (m + 1.0)
    z = s * s
    poly = 1.0 + z * (1.0 / 3.0 + z * (1.0 / 5.0 + z * (1.0 / 7.0 + z * (1.0 / 9.0))))
    return e.astype(jnp.float32) * LN2 + 2.0 * s * poly


def _fused_sc(x3d, mean, diag, covm2, batch, k):
    nw, idx_rows, _ = x3d.shape
    km1 = k - 1
    bs_per_w = batch // nw
    n_chunks = bs_per_w // CHUNK_B
    rows_per_chunk = CHUNK_B * k
    jrows = rows_per_chunk // LW
    pairs_per_chunk = CHUNK_B * km1
    n_groups = pairs_per_chunk // 16
    out_per_w = bs_per_w * km1
    info = plsc.get_sparse_core_info()
    assert nw == info.num_cores * info.num_subcores
    mesh = plsc.VectorSubcoreMesh(core_axis_name="c", subcore_axis_name="s")

    @functools.partial(
        pl.kernel,
        out_type=jax.ShapeDtypeStruct((batch * km1,), jnp.float32),
        mesh=mesh,
        scratch_types=[
            pltpu.VMEM((idx_rows, LW), jnp.int32),
            pltpu.VMEM((rows_per_chunk, DIM), jnp.float32),
            pltpu.VMEM((rows_per_chunk, DIM), jnp.float32),
            pltpu.VMEM((rows_per_chunk, 2 * DIM), jnp.float32),
            pltpu.VMEM((pairs_per_chunk,), jnp.float32),
            pltpu.SemaphoreType.DMA,
        ],
        compiler_params=pltpu.CompilerParams(
            use_tc_tiling_on_sc=False, needs_layout_passes=False),
    )
    def fused_k(x_hbm, mean_hbm, diag_hbm, covm_hbm, out_hbm,
                idx_v, mb, db, cb, ob, sem):
        wid = lax.axis_index("s") * info.num_cores + lax.axis_index("c")
        pltpu.sync_copy(x_hbm.at[wid], idx_v)

        def chunk_body(c, carry):
            cps = []
            for j in range(jrows):
                row = idx_v.at[c * jrows + j]
                dst = pl.ds(j * LW, LW)
                cps.append(pltpu.async_copy(mean_hbm.at[row], mb.at[dst], sem))
                cps.append(pltpu.async_copy(diag_hbm.at[row], db.at[dst], sem))
                cps.append(pltpu.async_copy(covm_hbm.at[row], cb.at[dst], sem))
            for cp in cps:
                cp.wait()

            def group_body(g, gcarry):
                p = g * 16 + lax.iota(jnp.int32, 16)
                b = jnp.right_shift(p * 3450, 16)
                ctx = p + b + 1
                anc = b * k
                one = jnp.ones((16,), jnp.float32)
                zero = jnp.zeros((16,), jnp.float32)
                m00 = one; m01 = zero; m11 = one
                q00 = one; q01 = zero; q11 = one
                term_diag = zero
                g_uu = zero; g_uv = zero; g_vv = zero
                s_acc = zero
                a_uu = zero; a_uv = zero; a_vu = zero; a_vv = zero
                dq = zero; p_u = zero; p_v = zero
                pl1 = one; ph1 = one; pl0 = one; ph0 = one
                for d in range(DIM):
                    cold = jnp.full((16,), d, jnp.int32)
                    col2 = jnp.full((16,), 2 * d, jnp.int32)
                    col2p = jnp.full((16,), 2 * d + 1, jnp.int32)
                    mu1 = plsc.load_gather(mb, [ctx, cold])
                    mu0 = plsc.load_gather(mb, [anc, cold])
                    d1 = plsc.load_gather(db, [ctx, cold])
                    d0 = plsc.load_gather(db, [anc, cold])
                    u1 = plsc.load_gather(cb, [ctx, col2])
                    v1 = plsc.load_gather(cb, [ctx, col2p])
                    c0u = plsc.load_gather(cb, [anc, col2])
                    c0v = plsc.load_gather(cb, [anc, col2p])
                    d1c = jnp.maximum(d1, 0.01)
                    d0c = jnp.maximum(d0, 0.01)
                    e1 = 1.0 / d1c
                    e0 = 1.0 / d0c
                    tu = u1 * e1
                    tv = v1 * e1
                    m00 = m00 + u1 * tu
                    m01 = m01 + u1 * tv
                    m11 = m11 + v1 * tv
                    t0u = c0u * e0
                    t0v = c0v * e0
                    q00 = q00 + c0u * t0u
                    q01 = q01 + c0u * t0v
                    q11 = q11 + c0v * t0v
                    term_diag = term_diag + d0c * e1
                    tmp = tu * d0c
                    g_uu = g_uu + tmp * tu
                    g_uv = g_uv + tmp * tv
                    g_vv = g_vv + (tv * d0c) * tv
                    s_acc = s_acc + (c0u * c0u + c0v * c0v) * e1
                    a_uu = a_uu + c0u * tu
                    a_uv = a_uv + c0u * tv
                    a_vu = a_vu + c0v * tu
                    a_vv = a_vv + c0v * tv
                    delta = mu1 - mu0
                    t = delta * e1
                    dq = dq + t * delta
                    p_u = p_u + t * u1
                    p_v = p_v + t * v1
                    if d < DIM // 2:
                        pl1 = pl1 * d1c
                        pl0 = pl0 * d0c
                    else:
                        ph1 = ph1 * d1c
                        ph0 = ph0 * d0c
                det1 = m00 * m11 - m01 * m01
                det0 = q00 * q11 - q01 * q01
                ld1 = _vlog(det1) + _vlog(pl1) + _vlog(ph1)
                ld0 = _vlog(det0) + _vlog(pl0) + _vlog(ph0)
                inv_det = 1.0 / det1

                def qf(a, bb):
                    return (m11 * a * a - 2.0 * m01 * a * bb + m00 * bb * bb) * inv_det

                gterm = (m11 * g_uu - 2.0 * m01 * g_uv + m00 * g_vv) * inv_det
                low = qf(a_uu, a_uv) + qf(a_vu, a_vv)
                tr = term_diag - gterm + s_acc - low
                quad = dq - qf(p_u, p_v)
                kl = 0.5 * (tr + quad - DIM + ld1 - ld0)
                ob[pl.ds(g * 16, 16)] = kl
                return gcarry

            lax.fori_loop(0, n_groups, group_body, 0)
            off = pl.multiple_of(wid * out_per_w + c * pairs_per_chunk, 16)
            pltpu.sync_copy(ob, out_hbm.at[pl.ds(off, pairs_per_chunk)])
            return carry

        lax.fori_loop(0, n_chunks, chunk_body, 0)

    return fused_k(x3d, mean, diag, covm2)


def kernel(x, mean, diag, covm):
    batch, k = x.shape
    nw = 32
    x3d = x.reshape(nw, -1, LW)
    covm2 = covm.reshape(covm.shape[0], DIM * RANK)
    flat = _fused_sc(x3d, mean, diag, covm2, batch, k)
    return flat.reshape(batch, k - 1)
